# baseline (speedup 1.0000x reference)
import jax
import jax.numpy as jnp
from jax.experimental import pallas as pl
from jax.experimental.pallas import tpu as pltpu

_BN = 512
_BK = 1024


def _assign_body(x_ref, xsq_ref, c_ref, csq_ref, out_ref, minval, minidx):
    j = pl.program_id(1)
    nk = pl.num_programs(1)
    x = x_ref[...]
    c = c_ref[...]
    cross = jax.lax.dot_general(
        x, c, dimension_numbers=(((1,), (1,)), ((), ())),
        preferred_element_type=jnp.float32)
    d2 = (xsq_ref[...] + csq_ref[0]) - 2.0 * cross
    d = jnp.sqrt(jnp.maximum(d2, 0.0))
    tmin = jnp.min(d, axis=1, keepdims=True)
    targ = jnp.argmin(d, axis=1).astype(jnp.int32)[:, None] + j * _BK

    @pl.when(j == 0)
    def _init():
        minval[...] = tmin
        minidx[...] = targ

    @pl.when(j != 0)
    def _accum():
        prev = minval[...]
        upd = tmin < prev
        minval[...] = jnp.where(upd, tmin, prev)
        minidx[...] = jnp.where(upd, targ, minidx[...])

    @pl.when(j == nk - 1)
    def _emit():
        out_ref[...] = minidx[...]


def kernel(x, centroids):
    n, d_ = x.shape
    k = centroids.shape[0]
    x_sq = jnp.sum(x * x, axis=1, keepdims=True)
    c_sq = jnp.sum(centroids * centroids, axis=1)
    kb = k // _BK
    csq_r = c_sq.reshape(kb, 1, _BK)
    out = pl.pallas_call(
        _assign_body,
        grid=(n // _BN, kb),
        in_specs=[
            pl.BlockSpec((_BN, d_), lambda i, j: (i, 0)),
            pl.BlockSpec((_BN, 1), lambda i, j: (i, 0)),
            pl.BlockSpec((_BK, d_), lambda i, j: (j, 0)),
            pl.BlockSpec((1, 1, _BK), lambda i, j: (j, 0, 0)),
        ],
        out_specs=pl.BlockSpec((_BN, 1), lambda i, j: (i, 0)),
        out_shape=jax.ShapeDtypeStruct((n, 1), jnp.int32),
        scratch_shapes=[
            pltpu.VMEM((_BN, 1), jnp.float32),
            pltpu.VMEM((_BN, 1), jnp.int32),
        ],
        compiler_params=pltpu.CompilerParams(
            dimension_semantics=("parallel", "arbitrary")),
    )(x, x_sq, centroids, csq_r)
    assignments = out[:, 0]
    return (centroids[None, :, :], assignments)


# speedup vs baseline: 1.2123x; 1.2123x over previous
import jax
import jax.numpy as jnp
from jax.experimental import pallas as pl
from jax.experimental.pallas import tpu as pltpu

_BN = 512
_BK = 1024


def _assign_body(x_ref, xsq_ref, c_ref, csq_ref, out_ref, runval, runidx):
    j = pl.program_id(1)
    nk = pl.num_programs(1)
    x2 = x_ref[...] * -2.0
    c = c_ref[...]
    cross = jax.lax.dot_general(
        x2, c, dimension_numbers=(((1,), (1,)), ((), ())),
        preferred_element_type=jnp.float32)
    t = xsq_ref[...] + csq_ref[0]
    d = jnp.sqrt(jnp.maximum(t + cross, 0.0))
    idx = jax.lax.broadcasted_iota(jnp.int32, d.shape, 1) + j * _BK

    @pl.when(j == 0)
    def _init():
        runval[...] = d
        runidx[...] = idx

    @pl.when(j != 0)
    def _accum():
        prev = runval[...]
        upd = d < prev
        runval[...] = jnp.minimum(d, prev)
        runidx[...] = jnp.where(upd, idx, runidx[...])

    @pl.when(j == nk - 1)
    def _emit():
        v = runval[...]
        m = jnp.min(v, axis=1, keepdims=True)
        big = jnp.int32(nk * _BK)
        win = jnp.min(jnp.where(v == m, runidx[...], big), axis=1)
        out_ref[...] = win[:, None]


def kernel(x, centroids):
    n, d_ = x.shape
    k = centroids.shape[0]
    x_sq = jnp.sum(x * x, axis=1, keepdims=True)
    c_sq = jnp.sum(centroids * centroids, axis=1)
    kb = k // _BK
    csq_r = c_sq.reshape(kb, 1, _BK)
    out = pl.pallas_call(
        _assign_body,
        grid=(n // _BN, kb),
        in_specs=[
            pl.BlockSpec((_BN, d_), lambda i, j: (i, 0)),
            pl.BlockSpec((_BN, 1), lambda i, j: (i, 0)),
            pl.BlockSpec((_BK, d_), lambda i, j: (j, 0)),
            pl.BlockSpec((1, 1, _BK), lambda i, j: (j, 0, 0)),
        ],
        out_specs=pl.BlockSpec((_BN, 1), lambda i, j: (i, 0)),
        out_shape=jax.ShapeDtypeStruct((n, 1), jnp.int32),
        scratch_shapes=[
            pltpu.VMEM((_BN, _BK), jnp.float32),
            pltpu.VMEM((_BN, _BK), jnp.int32),
        ],
        compiler_params=pltpu.CompilerParams(
            dimension_semantics=("parallel", "arbitrary")),
    )(x, x_sq, centroids, csq_r)
    assignments = out[:, 0]
    return (centroids[None, :, :], assignments)
